# baseline (speedup 1.0000x reference)
import functools

import jax
import jax.numpy as jnp
import numpy as np
from jax import lax
from jax.experimental import pallas as pl
from jax.experimental.pallas import tpu as pltpu
from jax.experimental.pallas import tpu_sc as plsc

_N = 15
_M = 16
_B = 32
_C = 64
_CH = 32
_H = 48
_W = 48
_HP = _CH * _N * _N
_HB = 7440
_OUT = 14625
_OUTP = 15360
_PL = _CH * _H * _W
_BIG = 1 << 20


def _consts():
    p = np.arange(_HP)
    cl = (p // (_N * _N)) * _H * _W
    ij = ((p % (_N * _N)) // _N) * 16 + p % _N
    pk = ((cl << 8) | ij).astype(np.int32)
    f = np.arange(240)
    mij = np.where(f < _N * _N, (f // _N) * 16 + f % _N, 0).astype(np.int32)
    return pk, mij


def _sc_body(table, xs_all, ys_all, pk, mij, out,
             xs_v, ys_v, pk_v, mij_v, hcol_v, mask_v, mout_v, plane, hb, sem):
    wid = lax.axis_index("s") * 2 + lax.axis_index("c")

    pltpu.sync_copy(xs_all.at[pl.ds(wid * _M, _M)], xs_v)
    pltpu.sync_copy(ys_all.at[pl.ds(wid * _M, _M)], ys_v)
    pltpu.sync_copy(pk, pk_v)
    pltpu.sync_copy(mij, mij_v)

    iv = lax.iota(jnp.int32, 16)
    plane[pl.ds(_PL, 16)] = jnp.zeros((16,), jnp.float32)

    def half_body(half, hcarry):
        base0 = (wid * _C + half * _CH) * _H * _W
        pltpu.sync_copy(table.at[pl.ds(base0, _PL)], plane.at[pl.ds(0, _PL)])

        def patch_body(t, carry):
            tb = t & 1

            @pl.when(t >= 2)
            def _drain():
                pltpu.make_async_copy(
                    table.at[pl.ds(0, _HP)], hb.at[tb, pl.ds(0, _HP)],
                    sem).wait()

            tv = jnp.full((16,), t, jnp.int32)
            xsb = plsc.load_gather(xs_v, [tv])
            ysb = plsc.load_gather(ys_v, [tv])
            o = ysb - 7

            colv = jnp.clip(o + iv, 0, _W - 1)
            colok = (o + iv >= 0) & (o + iv < _W) & (iv < _N)
            for i in range(_N):
                hrow = jnp.clip(xsb + (i - 7), 0, _H - 1) * _W
                rowok = (xsb >= 7 - i) & (xsb < _H + 7 - i)
                hcol_v[pl.ds(i * 16, 16)] = jnp.where(
                    colok & rowok, hrow + colv, jnp.int32(_BIG))
                mask_v[pl.ds(i * 16, 16)] = jnp.where(
                    colok & rowok, jnp.float32(1.0), jnp.float32(0.0))

            hbv = hb.at[tb]

            @plsc.parallel_loop(0, _HP // 16, unroll=8)
            def ext_body(p):
                sl = pl.ds(p * 16, 16)
                pkv = pk_v[sl]
                idx = (pkv >> 8) + plsc.load_gather(hcol_v, [pkv & 255])
                idx = jnp.minimum(idx, _PL)
                hbv[sl] = plsc.load_gather(plane, [idx])

            bm = wid * _M + t
            obase = bm * _OUTP + half * _HP
            pltpu.async_copy(hb.at[tb, pl.ds(0, _HP)],
                             out.at[pl.ds(obase, _HP)], sem)

            @pl.when(half == 1)
            def _mask_out():
                @plsc.parallel_loop(0, 15, unroll=5)
                def msk_body(q):
                    mout_v[pl.ds(q * 16, 16)] = plsc.load_gather(
                        mask_v, [mij_v[pl.ds(q * 16, 16)]])
                pltpu.sync_copy(mout_v.at[pl.ds(0, _N * _N)],
                                out.at[pl.ds(bm * _OUTP + 2 * _HP, _N * _N)])
            return carry

        lax.fori_loop(0, _M, patch_body, 0)
        for k in range(2):
            pltpu.make_async_copy(
                table.at[pl.ds(0, _HP)], hb.at[k, pl.ds(0, _HP)], sem).wait()
        return hcarry

    lax.fori_loop(0, 2, half_body, 0)


def kernel(x, x_cord, y_cord, one_player):
    if one_player is not None:
        start = _M * jnp.asarray(one_player, dtype=jnp.int32)
        x_cord = lax.dynamic_slice_in_dim(x_cord, start, _M, axis=1)
        y_cord = lax.dynamic_slice_in_dim(y_cord, start, _M, axis=1)
    xs_all = x_cord.reshape(-1).astype(jnp.int32)
    ys_all = y_cord.reshape(-1).astype(jnp.int32)
    table = x.reshape(_B * _C * _H * _W)

    pk, mij = _consts()

    mesh = plsc.VectorSubcoreMesh(core_axis_name="c", subcore_axis_name="s")
    sc = functools.partial(
        pl.kernel,
        mesh=mesh,
        compiler_params=pltpu.CompilerParams(
            needs_layout_passes=False, use_tc_tiling_on_sc=False,
            skip_device_barrier=True),
        out_type=jax.ShapeDtypeStruct((_B * _M * _OUTP,), jnp.float32),
        scratch_types=[
            pltpu.VMEM((_M,), jnp.int32),
            pltpu.VMEM((_M,), jnp.int32),
            pltpu.VMEM((_HP,), jnp.int32),
            pltpu.VMEM((240,), jnp.int32),
            pltpu.VMEM((240,), jnp.int32),
            pltpu.VMEM((240,), jnp.float32),
            pltpu.VMEM((240,), jnp.float32),
            pltpu.VMEM((_PL + 16,), jnp.float32),
            pltpu.VMEM((2, _HB), jnp.float32),
            pltpu.SemaphoreType.DMA,
        ],
    )(_sc_body)

    out = sc(table, xs_all, ys_all, jnp.asarray(pk), jnp.asarray(mij))

    def _trim_body(x_ref, o_ref):
        o_ref[...] = x_ref[...].reshape(8, _OUTP)[:, :_OUT]

    out = pl.pallas_call(
        _trim_body,
        grid=(_B * _M // 8,),
        in_specs=[pl.BlockSpec((8 * _OUTP,), lambda i: (i,))],
        out_specs=pl.BlockSpec((8, _OUT), lambda i: (i, 0)),
        out_shape=jax.ShapeDtypeStruct((_B * _M, _OUT), jnp.float32),
    )(out)
    return out.reshape(_B * _M, _C + 1, _N, _N)


# speedup vs baseline: 1.0350x; 1.0350x over previous
import functools

import jax
import jax.numpy as jnp
import numpy as np
from jax import lax
from jax.experimental import pallas as pl
from jax.experimental.pallas import tpu as pltpu
from jax.experimental.pallas import tpu_sc as plsc

_N = 15
_M = 16
_B = 32
_C = 64
_CH = 32
_H = 48
_W = 48
_HP = _CH * _N * _N
_HB = 7440
_OUT = 14625
_OUTP = 14632
_PL = _CH * _H * _W
_BIG = 1 << 20


def _consts():
    p = np.arange(_HP)
    cl = (p // (_N * _N)) * _H * _W
    ij = ((p % (_N * _N)) // _N) * 16 + p % _N
    pk = ((cl << 8) | ij).astype(np.int32)
    f = np.arange(240)
    mij = np.where(f < _N * _N, (f // _N) * 16 + f % _N, 0).astype(np.int32)
    return pk, mij


def _sc_body(table, xs_all, ys_all, pk, mij, out,
             xs_v, ys_v, pk_v, mij_v, hcol_v, mask_v, mout_v, plane, hb, sem):
    wid = lax.axis_index("s") * 2 + lax.axis_index("c")

    pltpu.sync_copy(xs_all.at[pl.ds(wid * _M, _M)], xs_v)
    pltpu.sync_copy(ys_all.at[pl.ds(wid * _M, _M)], ys_v)
    pltpu.sync_copy(pk, pk_v)
    pltpu.sync_copy(mij, mij_v)

    iv = lax.iota(jnp.int32, 16)
    plane[pl.ds(_PL, 16)] = jnp.zeros((16,), jnp.float32)

    def half_body(half, hcarry):
        base0 = (wid * _C + half * _CH) * _H * _W
        pltpu.sync_copy(table.at[pl.ds(base0, _PL)], plane.at[pl.ds(0, _PL)])

        def patch_body(t, carry):
            tb = t & 1

            @pl.when(t >= 2)
            def _drain():
                pltpu.make_async_copy(
                    table.at[pl.ds(0, _HP)], hb.at[tb, pl.ds(0, _HP)],
                    sem).wait()

            tv = jnp.full((16,), t, jnp.int32)
            xsb = plsc.load_gather(xs_v, [tv])
            ysb = plsc.load_gather(ys_v, [tv])
            o = ysb - 7

            colv = jnp.clip(o + iv, 0, _W - 1)
            colok = (o + iv >= 0) & (o + iv < _W) & (iv < _N)
            for i in range(_N):
                hrow = jnp.clip(xsb + (i - 7), 0, _H - 1) * _W
                rowok = (xsb >= 7 - i) & (xsb < _H + 7 - i)
                hcol_v[pl.ds(i * 16, 16)] = jnp.where(
                    colok & rowok, hrow + colv, jnp.int32(_BIG))
                mask_v[pl.ds(i * 16, 16)] = jnp.where(
                    colok & rowok, jnp.float32(1.0), jnp.float32(0.0))

            hbv = hb.at[tb]

            @plsc.parallel_loop(0, _HP // 16, unroll=8)
            def ext_body(p):
                sl = pl.ds(p * 16, 16)
                pkv = pk_v[sl]
                idx = (pkv >> 8) + plsc.load_gather(hcol_v, [pkv & 255])
                idx = jnp.minimum(idx, _PL)
                hbv[sl] = plsc.load_gather(plane, [idx])

            bm = wid * _M + t
            obase = bm * _OUTP + half * _HP
            pltpu.async_copy(hb.at[tb, pl.ds(0, _HP)],
                             out.at[pl.ds(obase, _HP)], sem)

            @pl.when(half == 1)
            def _mask_out():
                @plsc.parallel_loop(0, 15, unroll=5)
                def msk_body(q):
                    mout_v[pl.ds(q * 16, 16)] = plsc.load_gather(
                        mask_v, [mij_v[pl.ds(q * 16, 16)]])
                pltpu.sync_copy(mout_v.at[pl.ds(0, _N * _N)],
                                out.at[pl.ds(bm * _OUTP + 2 * _HP, _N * _N)])
            return carry

        lax.fori_loop(0, _M, patch_body, 0)
        for k in range(2):
            pltpu.make_async_copy(
                table.at[pl.ds(0, _HP)], hb.at[k, pl.ds(0, _HP)], sem).wait()
        return hcarry

    lax.fori_loop(0, 2, half_body, 0)


def kernel(x, x_cord, y_cord, one_player):
    if one_player is not None:
        start = _M * jnp.asarray(one_player, dtype=jnp.int32)
        x_cord = lax.dynamic_slice_in_dim(x_cord, start, _M, axis=1)
        y_cord = lax.dynamic_slice_in_dim(y_cord, start, _M, axis=1)
    xs_all = x_cord.reshape(-1).astype(jnp.int32)
    ys_all = y_cord.reshape(-1).astype(jnp.int32)
    table = x.reshape(_B * _C * _H * _W)

    pk, mij = _consts()

    mesh = plsc.VectorSubcoreMesh(core_axis_name="c", subcore_axis_name="s")
    sc = functools.partial(
        pl.kernel,
        mesh=mesh,
        compiler_params=pltpu.CompilerParams(
            needs_layout_passes=False, use_tc_tiling_on_sc=False,
            skip_device_barrier=True),
        out_type=jax.ShapeDtypeStruct((_B * _M * _OUTP,), jnp.float32),
        scratch_types=[
            pltpu.VMEM((_M,), jnp.int32),
            pltpu.VMEM((_M,), jnp.int32),
            pltpu.VMEM((_HP,), jnp.int32),
            pltpu.VMEM((240,), jnp.int32),
            pltpu.VMEM((240,), jnp.int32),
            pltpu.VMEM((240,), jnp.float32),
            pltpu.VMEM((240,), jnp.float32),
            pltpu.VMEM((_PL + 16,), jnp.float32),
            pltpu.VMEM((2, _HB), jnp.float32),
            pltpu.SemaphoreType.DMA,
        ],
    )(_sc_body)

    out = sc(table, xs_all, ys_all, jnp.asarray(pk), jnp.asarray(mij))

    out = out.reshape(_B * _M, _OUTP)[:, :_OUT]
    return out.reshape(_B * _M, _C + 1, _N, _N)
